# baseline (speedup 1.0000x reference)
import jax
import jax.numpy as jnp
from jax import lax
from jax.experimental import pallas as pl
from jax.experimental.pallas import tpu as pltpu
from jax.experimental.pallas import tpu_sc as plsc

B, N, D = 8, 2048, 128
NUM_CORES, NUM_SUBCORES = 2, 16
NW = NUM_CORES * NUM_SUBCORES
SUBC_PER_ROW = NW // B
SB_P = 64
SB_F = SB_P * D
NSB = (N // SB_P) // SUBC_PER_ROW
ROW_F = N * D
VEC = 16


def _sc_body(x_hbm, off_hbm, z_hbm, out_hbm, buf, offb, sem_in, sem_out):
    c = lax.axis_index("c")
    s = lax.axis_index("s")
    wid = c * NUM_SUBCORES + s
    b = wid % B
    q = wid // B
    row_base = b * ROW_F

    pltpu.sync_copy(off_hbm, offb.at[pl.ds(0, B + 1)])
    offv = offb[pl.ds(b, VEC)]
    nv = jnp.clip(offv[1] - offv[0], 0, N)

    zero = jnp.zeros((VEC,), jnp.float32)

    def sb_pos(k):
        return (q + SUBC_PER_ROW * k) * SB_P

    for k in range(NSB):
        pos = sb_pos(k)
        dst = out_hbm.at[pl.ds(row_base + pos * D, SB_F)]
        src = x_hbm.at[pl.ds(row_base + pos * D, SB_F)]

        @pl.when(pos + SB_P <= nv)
        def _(src=src, dst=dst):
            pltpu.async_copy(src, dst, sem_out)

        @pl.when(pos >= nv)
        def _(dst=dst):
            pltpu.async_copy(z_hbm, dst, sem_out)

        @pl.when(jnp.logical_and(pos < nv, nv < pos + SB_P))
        def _(src=src, dst=dst, pos=pos):
            pltpu.async_copy(src, buf, sem_in)
            pltpu.make_async_copy(src, buf, sem_in).wait()

            nvk = nv - pos

            def ztail(p, carry):
                for u in range(D // VEC):
                    buf[pl.ds(p * D + u * VEC, VEC)] = zero
                return carry

            lax.fori_loop(nvk, SB_P, ztail, 0)
            pltpu.async_copy(buf, dst, sem_out)

    for _k in range(NSB):
        pltpu.make_async_copy(z_hbm, out_hbm.at[pl.ds(0, SB_F)], sem_out).wait()


def kernel(x, x_offsets, all_timestamps, invalid_attn_mask):
    del all_timestamps, invalid_attn_mask
    xf = x.reshape(-1)
    off = x_offsets.astype(jnp.int32)
    zblock = jnp.zeros((SB_F,), jnp.float32)
    mesh = plsc.VectorSubcoreMesh(core_axis_name="c", subcore_axis_name="s")
    fn = pl.kernel(
        _sc_body,
        mesh=mesh,
        out_type=jax.ShapeDtypeStruct((B * N * D,), jnp.float32),
        scratch_types=[
            pltpu.VMEM((SB_F,), jnp.float32),
            pltpu.VMEM((32,), jnp.int32),
            pltpu.SemaphoreType.DMA,
            pltpu.SemaphoreType.DMA,
        ],
    )
    return fn(xf, off, zblock).reshape(B, N, D)


# speedup vs baseline: 9.8018x; 9.8018x over previous
import jax
import jax.numpy as jnp
from jax import lax
from jax.experimental import pallas as pl
from jax.experimental.pallas import tpu as pltpu
from jax.experimental.pallas import tpu_sc as plsc

B, N, D = 8, 2048, 128
NUM_CORES, NUM_SUBCORES = 2, 16
NW = NUM_CORES * NUM_SUBCORES
SUBC_PER_ROW = NW // B
SB_P = 32
SB_F = SB_P * D
NSB = (N // SB_P) // SUBC_PER_ROW
ROW_F = N * D
VEC = 16


def _sc_body(x_hbm, off_hbm, out_hbm, buf, zbuf, offb, sem_off, sem_in, sem_out):
    c = lax.axis_index("c")
    s = lax.axis_index("s")
    wid = c * NUM_SUBCORES + s
    b = wid % B
    q = wid // B
    row_base = b * ROW_F

    def sb_pos(k):
        return (q + SUBC_PER_ROW * k) * SB_P

    for k in range(NSB):
        pltpu.async_copy(
            x_hbm.at[pl.ds(row_base + sb_pos(k) * D, SB_F)],
            buf.at[pl.ds(k * SB_F, SB_F)],
            sem_in.at[k],
        )

    off_copy = pltpu.make_async_copy(off_hbm, offb.at[pl.ds(0, B + 1)], sem_off)
    off_copy.start()

    zero = jnp.zeros((VEC,), jnp.float32)

    def zfill(p, carry):
        for u in range(D // VEC):
            zbuf[pl.ds(p * D + u * VEC, VEC)] = zero
        return carry

    lax.fori_loop(0, SB_P, zfill, 0)

    off_copy.wait()
    offv = offb[pl.ds(b, VEC)]
    nv = jnp.clip(offv[1] - offv[0], 0, N)

    for k in range(NSB):
        @pl.when(sb_pos(k) >= nv)
        def _(k=k):
            pltpu.async_copy(
                zbuf, out_hbm.at[pl.ds(row_base + sb_pos(k) * D, SB_F)], sem_out
            )

    for k in range(NSB):
        @pl.when(sb_pos(k) < nv)
        def _(k=k):
            pltpu.make_async_copy(
                x_hbm.at[pl.ds(row_base + sb_pos(k) * D, SB_F)],
                buf.at[pl.ds(k * SB_F, SB_F)],
                sem_in.at[k],
            ).wait()

            nvk = jnp.minimum(nv - sb_pos(k), SB_P)

            def ztail(p, carry):
                for u in range(D // VEC):
                    buf[pl.ds(k * SB_F + p * D + u * VEC, VEC)] = zero
                return carry

            lax.fori_loop(nvk, SB_P, ztail, 0)

            pltpu.async_copy(
                buf.at[pl.ds(k * SB_F, SB_F)],
                out_hbm.at[pl.ds(row_base + sb_pos(k) * D, SB_F)],
                sem_out,
            )

    for k in range(NSB):
        pltpu.make_async_copy(
            zbuf, out_hbm.at[pl.ds(0, SB_F)], sem_out
        ).wait()

        @pl.when(sb_pos(k) >= nv)
        def _(k=k):
            pltpu.make_async_copy(
                x_hbm.at[pl.ds(row_base + sb_pos(k) * D, SB_F)],
                buf.at[pl.ds(k * SB_F, SB_F)],
                sem_in.at[k],
            ).wait()


def kernel(x, x_offsets, all_timestamps, invalid_attn_mask):
    del all_timestamps, invalid_attn_mask
    xf = x.reshape(-1)
    off = x_offsets.astype(jnp.int32)
    mesh = plsc.VectorSubcoreMesh(core_axis_name="c", subcore_axis_name="s")
    fn = pl.kernel(
        _sc_body,
        mesh=mesh,
        out_type=jax.ShapeDtypeStruct((B * N * D,), jnp.float32),
        scratch_types=[
            pltpu.VMEM((NSB * SB_F,), jnp.float32),
            pltpu.VMEM((SB_F,), jnp.float32),
            pltpu.VMEM((32,), jnp.int32),
            pltpu.SemaphoreType.DMA,
            pltpu.SemaphoreType.DMA((NSB,)),
            pltpu.SemaphoreType.DMA,
        ],
    )
    return fn(xf, off).reshape(B, N, D)


# speedup vs baseline: 10.4642x; 1.0676x over previous
import jax
import jax.numpy as jnp
from jax import lax
from jax.experimental import pallas as pl
from jax.experimental.pallas import tpu as pltpu
from jax.experimental.pallas import tpu_sc as plsc

B, N, D = 8, 2048, 128
NUM_CORES, NUM_SUBCORES = 2, 16
NW = NUM_CORES * NUM_SUBCORES
SUBC_PER_ROW = NW // B
SB_P = 128
SB_F = SB_P * D
NSB = (N // SB_P) // SUBC_PER_ROW
ROW_F = N * D
VEC = 16


def _sc_body(x_hbm, off_hbm, out_hbm, buf, zbuf, offb, sem_off, sem_in, sem_out):
    c = lax.axis_index("c")
    s = lax.axis_index("s")
    wid = c * NUM_SUBCORES + s
    b = wid % B
    q = wid // B
    row_base = b * ROW_F

    def sb_pos(k):
        return (q + SUBC_PER_ROW * k) * SB_P

    for k in range(NSB):
        pltpu.async_copy(
            x_hbm.at[pl.ds(row_base + sb_pos(k) * D, SB_F)],
            buf.at[pl.ds(k * SB_F, SB_F)],
            sem_in.at[k],
        )

    off_copy = pltpu.make_async_copy(off_hbm, offb.at[pl.ds(0, B + 1)], sem_off)
    off_copy.start()

    zero = jnp.zeros((VEC,), jnp.float32)

    def zfill(p, carry):
        for u in range(D // VEC):
            zbuf[pl.ds(p * D + u * VEC, VEC)] = zero
        return carry

    lax.fori_loop(0, SB_P, zfill, 0)

    off_copy.wait()
    offv = offb[pl.ds(b, VEC)]
    nv = jnp.clip(offv[1] - offv[0], 0, N)

    for k in range(NSB):
        @pl.when(sb_pos(k) >= nv)
        def _(k=k):
            pltpu.async_copy(
                zbuf, out_hbm.at[pl.ds(row_base + sb_pos(k) * D, SB_F)], sem_out
            )

    for k in range(NSB):
        @pl.when(sb_pos(k) < nv)
        def _(k=k):
            pltpu.make_async_copy(
                x_hbm.at[pl.ds(row_base + sb_pos(k) * D, SB_F)],
                buf.at[pl.ds(k * SB_F, SB_F)],
                sem_in.at[k],
            ).wait()

            nvk = jnp.minimum(nv - sb_pos(k), SB_P)

            def ztail(p, carry):
                for u in range(D // VEC):
                    buf[pl.ds(k * SB_F + p * D + u * VEC, VEC)] = zero
                return carry

            lax.fori_loop(nvk, SB_P, ztail, 0)

            pltpu.async_copy(
                buf.at[pl.ds(k * SB_F, SB_F)],
                out_hbm.at[pl.ds(row_base + sb_pos(k) * D, SB_F)],
                sem_out,
            )

    for k in range(NSB):
        pltpu.make_async_copy(
            zbuf, out_hbm.at[pl.ds(0, SB_F)], sem_out
        ).wait()

        @pl.when(sb_pos(k) >= nv)
        def _(k=k):
            pltpu.make_async_copy(
                x_hbm.at[pl.ds(row_base + sb_pos(k) * D, SB_F)],
                buf.at[pl.ds(k * SB_F, SB_F)],
                sem_in.at[k],
            ).wait()


def kernel(x, x_offsets, all_timestamps, invalid_attn_mask):
    del all_timestamps, invalid_attn_mask
    xf = x.reshape(-1)
    off = x_offsets.astype(jnp.int32)
    mesh = plsc.VectorSubcoreMesh(core_axis_name="c", subcore_axis_name="s")
    fn = pl.kernel(
        _sc_body,
        mesh=mesh,
        out_type=jax.ShapeDtypeStruct((B * N * D,), jnp.float32),
        scratch_types=[
            pltpu.VMEM((NSB * SB_F,), jnp.float32),
            pltpu.VMEM((SB_F,), jnp.float32),
            pltpu.VMEM((32,), jnp.int32),
            pltpu.SemaphoreType.DMA,
            pltpu.SemaphoreType.DMA((NSB,)),
            pltpu.SemaphoreType.DMA,
        ],
    )
    return fn(xf, off).reshape(B, N, D)
